# baseline (device time: 773380 ns/iter reference)
import jax
import jax.numpy as jnp
from jax import lax
from jax.experimental import pallas as pl
from jax.experimental.pallas import tpu as pltpu

M = 8192
D = 4096
M_HALF = M // 2
N_CHUNK = 32
CH = M_HALF // N_CHUNK


def kernel(partial, gamma):
    x = partial.reshape(M, D)
    g = gamma.reshape(1, D)

    def body(x_ref, g_ref, out_ref, sbuf, rbuf, av, ov, send_sems,
             recv_sems, sem_a, sem_s, sem_o):
        my_x = lax.axis_index("x")
        my_y = lax.axis_index("y")
        my_z = lax.axis_index("z")
        peer = (1 - my_x, my_y, my_z)

        barrier = pltpu.get_barrier_semaphore()
        pl.semaphore_signal(barrier, inc=1, device_id=peer,
                            device_id_type=pl.DeviceIdType.MESH)
        pl.semaphore_wait(barrier, 1)

        src_base = (1 - my_x) * M_HALF
        own_base = my_x * M_HALF

        def stage_and_send(c):
            slot = c % 2
            cp_s = pltpu.make_async_copy(
                x_ref.at[pl.ds(src_base + c * CH, CH), :], sbuf.at[slot],
                sem_s.at[slot])
            cp_s.start()
            cp_s.wait()
            r = pltpu.make_async_remote_copy(
                src_ref=sbuf.at[slot],
                dst_ref=rbuf.at[slot],
                send_sem=send_sems.at[slot],
                recv_sem=recv_sems.at[slot],
                device_id=peer,
                device_id_type=pl.DeviceIdType.MESH,
            )
            r.start()
            return r

        rdmas = {0: stage_and_send(0), 1: stage_and_send(1)}

        for c in range(N_CHUNK):
            slot = c % 2
            rdmas[c].wait_recv()
            cp_a = pltpu.make_async_copy(
                x_ref.at[pl.ds(own_base + c * CH, CH), :], av, sem_a)
            cp_a.start()
            cp_a.wait()
            y = av[...] + rbuf[slot, :, :]
            rms = jnp.sqrt(jnp.mean(y * y, axis=1, keepdims=True) + 1e-6)
            ov[...] = y / rms * g_ref[...]
            if c + 2 < N_CHUNK:
                pl.semaphore_signal(barrier, inc=1, device_id=peer,
                                    device_id_type=pl.DeviceIdType.MESH)
            cp_o = pltpu.make_async_copy(
                ov, out_ref.at[pl.ds(c * CH, CH), :], sem_o)
            cp_o.start()
            cp_o.wait()
            if c + 2 < N_CHUNK:
                rdmas[c].wait_send()
                pl.semaphore_wait(barrier, 1)
                rdmas[c + 2] = stage_and_send(c + 2)

        rdmas[N_CHUNK - 2].wait_send()
        rdmas[N_CHUNK - 1].wait_send()

    return pl.pallas_call(
        body,
        out_shape=jax.ShapeDtypeStruct((M_HALF, D), jnp.float32),
        in_specs=[
            pl.BlockSpec(memory_space=pl.ANY),
            pl.BlockSpec(memory_space=pltpu.VMEM),
        ],
        out_specs=pl.BlockSpec(memory_space=pl.ANY),
        scratch_shapes=[
            pltpu.VMEM((2, CH, D), jnp.float32),
            pltpu.VMEM((2, CH, D), jnp.float32),
            pltpu.VMEM((CH, D), jnp.float32),
            pltpu.VMEM((CH, D), jnp.float32),
            pltpu.SemaphoreType.DMA((2,)),
            pltpu.SemaphoreType.DMA((2,)),
            pltpu.SemaphoreType.DMA,
            pltpu.SemaphoreType.DMA((2,)),
            pltpu.SemaphoreType.DMA,
        ],
        compiler_params=pltpu.CompilerParams(collective_id=0),
    )(x, g)
